# baseline (speedup 1.0000x reference)
import functools

import jax
import jax.numpy as jnp
from jax import lax
from jax.experimental import pallas as pl
from jax.experimental.pallas import tpu as pltpu
from jax.experimental.pallas import tpu_sc as plsc

_N = 50000
_E = 800000
_N_PAD = 50176
_LIN = _N_PAD * 16 // 128
_BLK = 128
_GRID = _LIN // _BLK
_NTILES = 32
_C = 128
_CH = 196
_K = 14
_E_PAD = _NTILES * _CH * _C
_RPT = _N_PAD // 16
_ZR = 112


@functools.cache
def _make_sc_agg(w):
    mesh = plsc.VectorSubcoreMesh(core_axis_name="c", subcore_axis_name="s")

    @functools.partial(
        pl.kernel,
        out_type=jax.ShapeDtypeStruct((2, _N_PAD, w), jnp.float32),
        mesh=mesh,
        scratch_types=[
            pltpu.VMEM((_CH, _C), jnp.int32),
            pltpu.VMEM((_CH, _C), jnp.int32),
            pltpu.VMEM((_K, _C, w), jnp.float32),
            pltpu.VMEM((_ZR, w), jnp.float32),
            pltpu.VMEM_SHARED((_N_PAD, w), jnp.float32),
            pltpu.SemaphoreType.DMA,
            pltpu.SemaphoreType.DMA,
        ],
        compiler_params=pltpu.CompilerParams(use_tc_tiling_on_sc=False),
    )
    def sc_agg(m_hbm, edges_hbm, out_hbm,
               src_v, dst_v, rows_v, zbuf, acc, gsem, ssem):
        c = lax.axis_index("c")
        s = lax.axis_index("s")
        wid = c * 16 + s
        pltpu.sync_copy(edges_hbm.at[0, wid], src_v)
        pltpu.sync_copy(edges_hbm.at[1, wid], dst_v)
        zv = jnp.zeros((16,), jnp.float32)
        for r in range(_ZR):
            zbuf[r, pl.ds(0, 16)] = zv

        def zero_chunk(j, carry):
            pltpu.sync_copy(zbuf, acc.at[pl.ds(s * _RPT + j * _ZR, _ZR)])
            return carry

        lax.fori_loop(0, _RPT // _ZR, zero_chunk, 0)
        plsc.subcore_barrier()

        def group(g, carry):
            j0 = g * _K
            gathers = []
            for b in range(_K):
                gathers.append(pltpu.async_copy(
                    m_hbm.at[src_v.at[j0 + b]], rows_v.at[b], gsem))
            scatters = []
            for b in range(_K):
                gathers[b].wait()
                scatters.append(pltpu.async_copy(
                    rows_v.at[b], acc.at[dst_v.at[j0 + b]], ssem, add=True))
            for b in range(_K):
                scatters[b].wait()
            return carry

        lax.fori_loop(0, _CH // _K, group, 0)
        plsc.subcore_barrier()
        pltpu.sync_copy(acc.at[pl.ds(s * _RPT, _RPT)],
                        out_hbm.at[c, pl.ds(s * _RPT, _RPT)])

    return sc_agg


def _dot(x, w):
    return lax.dot_general(x, w, (((1,), (0,)), ((), ())),
                           preferred_element_type=jnp.float32)


def _full(shape):
    return pl.BlockSpec(shape, lambda i: (0,) * len(shape))


def _rows(w):
    return pl.BlockSpec((_BLK, w), lambda i: (i, 0))


def _agg_spec():
    return pl.BlockSpec((2, _BLK, 128), lambda i: (0, i, 0))


def _lin_struct(w=128):
    return jax.ShapeDtypeStruct((_LIN, w), jnp.float32)


def _tc_pre_body(z_ref, wa_ref, wb_ref, oa_ref, ob_ref):
    z = z_ref[...]
    oa_ref[...] = _dot(z, wa_ref[...])
    ob_ref[...] = _dot(z, wb_ref[...])


def _tc_pre(z8, BD1a, BD1b):
    return pl.pallas_call(
        _tc_pre_body,
        grid=(_GRID,),
        in_specs=[_rows(512), _full((512, 128)), _full((512, 128))],
        out_specs=[_rows(128), _rows(128)],
        out_shape=[_lin_struct(), _lin_struct()],
    )(z8, BD1a, BD1b)


def _tc_root_body(z_ref, wra_ref, wrb_ref, ra_ref, rb_ref):
    z = z_ref[...]
    ra_ref[...] = _dot(z, wra_ref[...])
    rb_ref[...] = _dot(z, wrb_ref[...])


def _tc_root(z8, BDra, BDrb):
    return pl.pallas_call(
        _tc_root_body,
        grid=(_GRID,),
        in_specs=[_rows(512), _full((512, 128)), _full((512, 128))],
        out_specs=[_rows(128), _rows(128)],
        out_shape=[_lin_struct(), _lin_struct()],
    )(z8, BDra, BDrb)


def _tc_mid1_body(aa_ref, ab_ref, ra_ref, rb_ref, ba_ref, bb_ref,
                  w2a_ref, w2b_ref, w2ra_ref, w2rb_ref, m2_ref, r2_ref):
    ha = jnp.maximum(aa_ref[0] + aa_ref[1] + ba_ref[...] + ra_ref[...], 0.0)
    hb = jnp.maximum(ab_ref[0] + ab_ref[1] + bb_ref[...] + rb_ref[...], 0.0)
    m2_ref[...] = _dot(ha, w2a_ref[...]) + _dot(hb, w2b_ref[...])
    r2_ref[...] = _dot(ha, w2ra_ref[...]) + _dot(hb, w2rb_ref[...])


def _tc_mid1(agg1a, agg1b, r1a, r1b, b1a_t, b1b_t,
             BD2a, BD2b, BD2ra, BD2rb):
    return pl.pallas_call(
        _tc_mid1_body,
        grid=(_GRID,),
        in_specs=[_agg_spec(), _agg_spec(), _rows(128), _rows(128),
                  _full((1, 128)), _full((1, 128)),
                  _full((128, 128)), _full((128, 128)),
                  _full((128, 128)), _full((128, 128))],
        out_specs=[_rows(128), _rows(128)],
        out_shape=[_lin_struct(), _lin_struct()],
    )(agg1a, agg1b, r1a, r1b, b1a_t, b1b_t, BD2a, BD2b, BD2ra, BD2rb)


def _tc_mid2_body(a_ref, r2_ref, b_ref, h2_ref):
    h2_ref[...] = jnp.maximum(a_ref[0] + a_ref[1] + b_ref[...] + r2_ref[...],
                              0.0)


def _tc_mid2(agg2, r2, b2_t):
    return pl.pallas_call(
        _tc_mid2_body,
        grid=(_GRID,),
        in_specs=[_agg_spec(), _rows(128), _full((1, 128))],
        out_specs=_rows(128),
        out_shape=_lin_struct(),
    )(agg2, r2, b2_t)


def _tc_finroot_body(h2_ref, w3r_ref, o_ref):
    o_ref[...] = _dot(h2_ref[...], w3r_ref[...])


def _tc_finroot(h2, BD3r):
    return pl.pallas_call(
        _tc_finroot_body,
        grid=(_GRID,),
        in_specs=[_rows(128), _full((128, 16))],
        out_specs=_rows(16),
        out_shape=_lin_struct(16),
    )(h2, BD3r)


def _tc_fin_body(a_ref, tr_ref, w3_ref, b_ref, o_ref):
    t = _dot(a_ref[0] + a_ref[1], w3_ref[...]) + b_ref[...] + tr_ref[...]
    col = lax.broadcasted_iota(jnp.int32, t.shape, 1)
    tl = jnp.concatenate([t[:, 1:], t[:, :1]], axis=1)
    tr = jnp.concatenate([t[:, -1:], t[:, :-1]], axis=1)
    other = jnp.where(col % 2 == 0, tl, tr)
    o_ref[...] = 1.0 / (1.0 + jnp.exp(other - t))


def _tc_fin(agg3, tro, BD3, b3_t):
    return pl.pallas_call(
        _tc_fin_body,
        grid=(_GRID,),
        in_specs=[_agg_spec(), _rows(16), _full((128, 16)), _full((1, 16))],
        out_specs=_rows(16),
        out_shape=_lin_struct(16),
    )(agg3, tro, BD3, b3_t)


def kernel(z, edge_index, W1_rel, b1, W1_root, W2_rel, b2, W2_root,
           W3_rel, b3, W3_root):
    f32 = jnp.float32
    e_pad = jnp.pad(edge_index, ((0, 0), (0, _E_PAD - _E)),
                    constant_values=_N).reshape(2, _NTILES, _CH, _C)
    z8 = jnp.pad(z.reshape(_N // 8, 512), ((0, _LIN - _N // 8), (0, 0)))

    I8 = jnp.eye(8, dtype=f32)
    kron = jnp.kron
    BD1a = kron(I8, W1_rel[:16].T)
    BD1b = kron(I8, W1_rel[16:].T)
    BDra = kron(I8, W1_root.T[:, :16])
    BDrb = kron(I8, W1_root.T[:, 16:])
    W2T = W2_rel.T
    W2rT = W2_root.T
    BD2a = kron(I8, W2T[:16])
    BD2b = kron(I8, W2T[16:])
    BD2ra = kron(I8, W2rT[:16])
    BD2rb = kron(I8, W2rT[16:])
    BD3 = kron(I8, W3_rel.T)
    BD3r = kron(I8, W3_root.T)
    b1a_t = jnp.tile(b1[:16], 8).reshape(1, 128)
    b1b_t = jnp.tile(b1[16:], 8).reshape(1, 128)
    b2_t = jnp.tile(b2, 8).reshape(1, 128)
    b3_t = jnp.tile(b3, 8).reshape(1, 16)

    sc16 = _make_sc_agg(16)

    big = lambda m: jnp.pad(m.reshape(_N_PAD, 16), ((0, 2 * _N_PAD), (0, 0)))
    m1a, m1b = _tc_pre(z8, BD1a, BD1b)
    agg1a = sc16(big(m1a), e_pad)
    agg1b = sc16(big(m1b), e_pad)
    r1a, r1b = _tc_root(z8, BDra, BDrb)
    m2, r2 = _tc_mid1(agg1a.reshape(2, _LIN, 128), agg1b.reshape(2, _LIN, 128),
                      r1a, r1b, b1a_t, b1b_t, BD2a, BD2b, BD2ra, BD2rb)
    agg2 = sc16(big(m2), e_pad)
    h2 = _tc_mid2(agg2.reshape(2, _LIN, 128), r2, b2_t)
    agg3 = sc16(big(h2), e_pad)
    tro = _tc_finroot(h2, BD3r)
    out = _tc_fin(agg3.reshape(2, _LIN, 128), tro, BD3, b3_t)
    return out[:_N // 8].reshape(_N, 2)


# speedup vs baseline: 1.6302x; 1.6302x over previous
import functools

import jax
import jax.numpy as jnp
from jax import lax
from jax.experimental import pallas as pl
from jax.experimental.pallas import tpu as pltpu
from jax.experimental.pallas import tpu_sc as plsc

_N = 50000
_E = 800000
_N_PAD = 50176
_LIN = _N_PAD * 16 // 128
_BLK = 128
_GRID = _LIN // _BLK
_NTILES = 32
_C = 128
_CH = 196
_K = 14
_E_PAD = _NTILES * _CH * _C
_RPT = _N_PAD // 16
_ZR = 112


@functools.cache
def _make_sc_agg(w):
    mesh = plsc.VectorSubcoreMesh(core_axis_name="c", subcore_axis_name="s")

    @functools.partial(
        pl.kernel,
        out_type=jax.ShapeDtypeStruct((2, _N_PAD, w), jnp.float32),
        mesh=mesh,
        scratch_types=[
            pltpu.VMEM((_CH, _C), jnp.int32),
            pltpu.VMEM((_CH, _C), jnp.int32),
            pltpu.VMEM((_K, _C, w), jnp.float32),
            pltpu.VMEM((_ZR, w), jnp.float32),
            pltpu.VMEM_SHARED((_N_PAD, w), jnp.float32),
            pltpu.SemaphoreType.DMA,
            pltpu.SemaphoreType.DMA,
        ],
        compiler_params=pltpu.CompilerParams(use_tc_tiling_on_sc=False),
    )
    def sc_agg(m_hbm, edges_hbm, out_hbm,
               src_v, dst_v, rows_v, zbuf, acc, gsem, ssem):
        c = lax.axis_index("c")
        s = lax.axis_index("s")
        wid = c * 16 + s
        pltpu.sync_copy(edges_hbm.at[0, wid], src_v)
        pltpu.sync_copy(edges_hbm.at[1, wid], dst_v)
        zv = jnp.zeros((16,), jnp.float32)
        for r in range(_ZR):
            zbuf[r, pl.ds(0, 16)] = zv

        def zero_chunk(j, carry):
            pltpu.sync_copy(zbuf, acc.at[pl.ds(s * _RPT + j * _ZR, _ZR)])
            return carry

        lax.fori_loop(0, _RPT // _ZR, zero_chunk, 0)
        plsc.subcore_barrier()

        def group(g, carry):
            j0 = g * _K
            gathers = []
            for b in range(_K):
                gathers.append(pltpu.async_copy(
                    m_hbm.at[src_v.at[j0 + b]], rows_v.at[b], gsem))
            scatters = []
            for b in range(_K):
                gathers[b].wait()
                scatters.append(pltpu.async_copy(
                    rows_v.at[b], acc.at[dst_v.at[j0 + b]], ssem, add=True))
            for b in range(_K):
                scatters[b].wait()
            return carry

        lax.fori_loop(0, _CH // _K, group, 0)
        plsc.subcore_barrier()
        pltpu.sync_copy(acc.at[pl.ds(s * _RPT, _RPT)],
                        out_hbm.at[c, pl.ds(s * _RPT, _RPT)])

    return sc_agg


def _dot(x, w):
    return lax.dot_general(x, w, (((1,), (0,)), ((), ())),
                           preferred_element_type=jnp.float32)


def _full(shape):
    return pl.BlockSpec(shape, lambda i: (0,) * len(shape))


def _rows(w):
    return pl.BlockSpec((_BLK, w), lambda i: (i, 0))


def _agg_spec():
    return pl.BlockSpec((2, _BLK, 128), lambda i: (0, i, 0))


def _lin_struct(w=128):
    return jax.ShapeDtypeStruct((_LIN, w), jnp.float32)


def _tc_pre_body(z_ref, wa_ref, wb_ref, oa_ref, ob_ref):
    z = z_ref[...]
    oa_ref[...] = _dot(z, wa_ref[...])
    ob_ref[...] = _dot(z, wb_ref[...])


def _tc_pre(z8, BD1a, BD1b):
    return pl.pallas_call(
        _tc_pre_body,
        grid=(_GRID,),
        in_specs=[_rows(512), _full((512, 128)), _full((512, 128))],
        out_specs=[_rows(128), _rows(128)],
        out_shape=[_lin_struct(), _lin_struct()],
    )(z8, BD1a, BD1b)


def _tc_root_body(z_ref, wra_ref, wrb_ref, ra_ref, rb_ref):
    z = z_ref[...]
    ra_ref[...] = _dot(z, wra_ref[...])
    rb_ref[...] = _dot(z, wrb_ref[...])


def _tc_root(z8, BDra, BDrb):
    return pl.pallas_call(
        _tc_root_body,
        grid=(_GRID,),
        in_specs=[_rows(512), _full((512, 128)), _full((512, 128))],
        out_specs=[_rows(128), _rows(128)],
        out_shape=[_lin_struct(), _lin_struct()],
    )(z8, BDra, BDrb)


def _tc_mid1_body(aa_ref, ab_ref, ra_ref, rb_ref, ba_ref, bb_ref,
                  w2a_ref, w2b_ref, w2ra_ref, w2rb_ref, m2_ref, r2_ref):
    ha = jnp.maximum(aa_ref[0] + aa_ref[1] + ba_ref[...] + ra_ref[...], 0.0)
    hb = jnp.maximum(ab_ref[0] + ab_ref[1] + bb_ref[...] + rb_ref[...], 0.0)
    m2_ref[...] = _dot(ha, w2a_ref[...]) + _dot(hb, w2b_ref[...])
    r2_ref[...] = _dot(ha, w2ra_ref[...]) + _dot(hb, w2rb_ref[...])


def _tc_mid1(agg1a, agg1b, r1a, r1b, b1a_t, b1b_t,
             BD2a, BD2b, BD2ra, BD2rb):
    return pl.pallas_call(
        _tc_mid1_body,
        grid=(_GRID,),
        in_specs=[_agg_spec(), _agg_spec(), _rows(128), _rows(128),
                  _full((1, 128)), _full((1, 128)),
                  _full((128, 128)), _full((128, 128)),
                  _full((128, 128)), _full((128, 128))],
        out_specs=[_rows(128), _rows(128)],
        out_shape=[_lin_struct(), _lin_struct()],
    )(agg1a, agg1b, r1a, r1b, b1a_t, b1b_t, BD2a, BD2b, BD2ra, BD2rb)


def _tc_mid2_body(a_ref, r2_ref, b_ref, h2_ref):
    h2_ref[...] = jnp.maximum(a_ref[0] + a_ref[1] + b_ref[...] + r2_ref[...],
                              0.0)


def _tc_mid2(agg2, r2, b2_t):
    return pl.pallas_call(
        _tc_mid2_body,
        grid=(_GRID,),
        in_specs=[_agg_spec(), _rows(128), _full((1, 128))],
        out_specs=_rows(128),
        out_shape=_lin_struct(),
    )(agg2, r2, b2_t)


def _tc_finroot_body(h2_ref, w3r_ref, o_ref):
    o_ref[...] = _dot(h2_ref[...], w3r_ref[...])


def _tc_finroot(h2, BD3r):
    return pl.pallas_call(
        _tc_finroot_body,
        grid=(_GRID,),
        in_specs=[_rows(128), _full((128, 16))],
        out_specs=_rows(16),
        out_shape=_lin_struct(16),
    )(h2, BD3r)


def _tc_fin_body(a_ref, tr_ref, w3_ref, b_ref, o_ref):
    t = _dot(a_ref[0] + a_ref[1], w3_ref[...]) + b_ref[...] + tr_ref[...]
    col = lax.broadcasted_iota(jnp.int32, t.shape, 1)
    tl = jnp.concatenate([t[:, 1:], t[:, :1]], axis=1)
    tr = jnp.concatenate([t[:, -1:], t[:, :-1]], axis=1)
    other = jnp.where(col % 2 == 0, tl, tr)
    o_ref[...] = 1.0 / (1.0 + jnp.exp(other - t))


def _tc_fin(agg3, tro, BD3, b3_t):
    return pl.pallas_call(
        _tc_fin_body,
        grid=(_GRID,),
        in_specs=[_agg_spec(), _rows(16), _full((128, 16)), _full((1, 16))],
        out_specs=_rows(16),
        out_shape=_lin_struct(16),
    )(agg3, tro, BD3, b3_t)


def kernel(z, edge_index, W1_rel, b1, W1_root, W2_rel, b2, W2_root,
           W3_rel, b3, W3_root):
    f32 = jnp.float32
    e_pad = jnp.pad(edge_index, ((0, 0), (0, _E_PAD - _E)),
                    constant_values=_N).reshape(2, _NTILES, _CH, _C)
    z8 = jnp.pad(z.reshape(_N // 8, 512), ((0, _LIN - _N // 8), (0, 0)))

    I8 = jnp.eye(8, dtype=f32)
    kron = jnp.kron
    BD1a = kron(I8, W1_rel[:16].T)
    BD1b = kron(I8, W1_rel[16:].T)
    BDra = kron(I8, W1_root.T[:, :16])
    BDrb = kron(I8, W1_root.T[:, 16:])
    W2T = W2_rel.T
    W2rT = W2_root.T
    BD2a = kron(I8, W2T[:16])
    BD2b = kron(I8, W2T[16:])
    BD2ra = kron(I8, W2rT[:16])
    BD2rb = kron(I8, W2rT[16:])
    BD3 = kron(I8, W3_rel.T)
    BD3r = kron(I8, W3_root.T)
    b1a_t = jnp.tile(b1[:16], 8).reshape(1, 128)
    b1b_t = jnp.tile(b1[16:], 8).reshape(1, 128)
    b2_t = jnp.tile(b2, 8).reshape(1, 128)
    b3_t = jnp.tile(b3, 8).reshape(1, 16)

    sc16 = _make_sc_agg(16)

    m1a, m1b = _tc_pre(z8, BD1a, BD1b)
    agg1a = sc16(m1a.reshape(_N_PAD, 16), e_pad)
    agg1b = sc16(m1b.reshape(_N_PAD, 16), e_pad)
    r1a, r1b = _tc_root(z8, BDra, BDrb)
    m2, r2 = _tc_mid1(agg1a.reshape(2, _LIN, 128), agg1b.reshape(2, _LIN, 128),
                      r1a, r1b, b1a_t, b1b_t, BD2a, BD2b, BD2ra, BD2rb)
    agg2 = sc16(m2.reshape(_N_PAD, 16), e_pad)
    h2 = _tc_mid2(agg2.reshape(2, _LIN, 128), r2, b2_t)
    agg3 = sc16(h2.reshape(_N_PAD, 16), e_pad)
    tro = _tc_finroot(h2, BD3r)
    out = _tc_fin(agg3.reshape(2, _LIN, 128), tro, BD3, b3_t)
    return out[:_N // 8].reshape(_N, 2)


# speedup vs baseline: 1.8936x; 1.1616x over previous
import functools

import jax
import jax.numpy as jnp
from jax import lax
from jax.experimental import pallas as pl
from jax.experimental.pallas import tpu as pltpu
from jax.experimental.pallas import tpu_sc as plsc

_N = 50000
_E = 800000
_N_PAD = 50176
_LIN = _N_PAD * 16 // 128
_BLK = 448
_GRID = _LIN // _BLK
_NTILES = 32
_C = 128
_CH = 196
_K = 14
_E_PAD = _NTILES * _CH * _C
_RPT = _N_PAD // 16
_ZR = 112


@functools.cache
def _make_sc_agg(w):
    mesh = plsc.VectorSubcoreMesh(core_axis_name="c", subcore_axis_name="s")

    @functools.partial(
        pl.kernel,
        out_type=jax.ShapeDtypeStruct((2, _N_PAD, w), jnp.float32),
        mesh=mesh,
        scratch_types=[
            pltpu.VMEM((_CH, _C), jnp.int32),
            pltpu.VMEM((_CH, _C), jnp.int32),
            pltpu.VMEM((_K, _C, w), jnp.float32),
            pltpu.VMEM((_ZR, w), jnp.float32),
            pltpu.VMEM_SHARED((_N_PAD, w), jnp.float32),
            pltpu.SemaphoreType.DMA,
            pltpu.SemaphoreType.DMA,
        ],
        compiler_params=pltpu.CompilerParams(use_tc_tiling_on_sc=False),
    )
    def sc_agg(m_hbm, edges_hbm, out_hbm,
               src_v, dst_v, rows_v, zbuf, acc, gsem, ssem):
        c = lax.axis_index("c")
        s = lax.axis_index("s")
        wid = c * 16 + s
        pltpu.sync_copy(edges_hbm.at[0, wid], src_v)
        pltpu.sync_copy(edges_hbm.at[1, wid], dst_v)
        zv = jnp.zeros((16,), jnp.float32)
        for r in range(_ZR):
            zbuf[r, pl.ds(0, 16)] = zv

        def zero_chunk(j, carry):
            pltpu.sync_copy(zbuf, acc.at[pl.ds(s * _RPT + j * _ZR, _ZR)])
            return carry

        lax.fori_loop(0, _RPT // _ZR, zero_chunk, 0)
        plsc.subcore_barrier()

        def group(g, carry):
            j0 = g * _K
            gathers = []
            for b in range(_K):
                gathers.append(pltpu.async_copy(
                    m_hbm.at[src_v.at[j0 + b]], rows_v.at[b], gsem))
            scatters = []
            for b in range(_K):
                gathers[b].wait()
                scatters.append(pltpu.async_copy(
                    rows_v.at[b], acc.at[dst_v.at[j0 + b]], ssem, add=True))
            for b in range(_K):
                scatters[b].wait()
            return carry

        lax.fori_loop(0, _CH // _K, group, 0)
        plsc.subcore_barrier()
        pltpu.sync_copy(acc.at[pl.ds(s * _RPT, _RPT)],
                        out_hbm.at[c, pl.ds(s * _RPT, _RPT)])

    return sc_agg


def _dot(x, w):
    return lax.dot_general(x, w, (((1,), (0,)), ((), ())),
                           preferred_element_type=jnp.float32)


def _full(shape):
    return pl.BlockSpec(shape, lambda i: (0,) * len(shape))


def _rows(w):
    return pl.BlockSpec((_BLK, w), lambda i: (i, 0))


def _agg_spec():
    return pl.BlockSpec((2, _BLK, 128), lambda i: (0, i, 0))


def _lin_struct(w=128):
    return jax.ShapeDtypeStruct((_LIN, w), jnp.float32)


def _tc_pre_body(z_ref, wa_ref, wb_ref, oa_ref, ob_ref):
    z = z_ref[...]
    oa_ref[...] = _dot(z, wa_ref[...])
    ob_ref[...] = _dot(z, wb_ref[...])


def _tc_pre(z8, BD1a, BD1b):
    return pl.pallas_call(
        _tc_pre_body,
        grid=(_GRID,),
        in_specs=[_rows(512), _full((512, 128)), _full((512, 128))],
        out_specs=[_rows(128), _rows(128)],
        out_shape=[_lin_struct(), _lin_struct()],
    )(z8, BD1a, BD1b)


def _tc_root_body(z_ref, wra_ref, wrb_ref, ra_ref, rb_ref):
    z = z_ref[...]
    ra_ref[...] = _dot(z, wra_ref[...])
    rb_ref[...] = _dot(z, wrb_ref[...])


def _tc_root(z8, BDra, BDrb):
    return pl.pallas_call(
        _tc_root_body,
        grid=(_GRID,),
        in_specs=[_rows(512), _full((512, 128)), _full((512, 128))],
        out_specs=[_rows(128), _rows(128)],
        out_shape=[_lin_struct(), _lin_struct()],
    )(z8, BDra, BDrb)


def _tc_mid1_body(aa_ref, ab_ref, ra_ref, rb_ref, ba_ref, bb_ref,
                  w2a_ref, w2b_ref, w2ra_ref, w2rb_ref, m2_ref, r2_ref):
    ha = jnp.maximum(aa_ref[0] + aa_ref[1] + ba_ref[...] + ra_ref[...], 0.0)
    hb = jnp.maximum(ab_ref[0] + ab_ref[1] + bb_ref[...] + rb_ref[...], 0.0)
    m2_ref[...] = _dot(ha, w2a_ref[...]) + _dot(hb, w2b_ref[...])
    r2_ref[...] = _dot(ha, w2ra_ref[...]) + _dot(hb, w2rb_ref[...])


def _tc_mid1(agg1a, agg1b, r1a, r1b, b1a_t, b1b_t,
             BD2a, BD2b, BD2ra, BD2rb):
    return pl.pallas_call(
        _tc_mid1_body,
        grid=(_GRID,),
        in_specs=[_agg_spec(), _agg_spec(), _rows(128), _rows(128),
                  _full((1, 128)), _full((1, 128)),
                  _full((128, 128)), _full((128, 128)),
                  _full((128, 128)), _full((128, 128))],
        out_specs=[_rows(128), _rows(128)],
        out_shape=[_lin_struct(), _lin_struct()],
    )(agg1a, agg1b, r1a, r1b, b1a_t, b1b_t, BD2a, BD2b, BD2ra, BD2rb)


def _tc_mid2_body(a_ref, r2_ref, b_ref, h2_ref):
    h2_ref[...] = jnp.maximum(a_ref[0] + a_ref[1] + b_ref[...] + r2_ref[...],
                              0.0)


def _tc_mid2(agg2, r2, b2_t):
    return pl.pallas_call(
        _tc_mid2_body,
        grid=(_GRID,),
        in_specs=[_agg_spec(), _rows(128), _full((1, 128))],
        out_specs=_rows(128),
        out_shape=_lin_struct(),
    )(agg2, r2, b2_t)


def _tc_finroot_body(h2_ref, w3r_ref, o_ref):
    o_ref[...] = _dot(h2_ref[...], w3r_ref[...])


def _tc_finroot(h2, BD3r):
    return pl.pallas_call(
        _tc_finroot_body,
        grid=(_GRID,),
        in_specs=[_rows(128), _full((128, 16))],
        out_specs=_rows(16),
        out_shape=_lin_struct(16),
    )(h2, BD3r)


def _tc_fin_body(a_ref, tr_ref, w3_ref, b_ref, o_ref):
    t = _dot(a_ref[0] + a_ref[1], w3_ref[...]) + b_ref[...] + tr_ref[...]
    col = lax.broadcasted_iota(jnp.int32, t.shape, 1)
    tl = jnp.concatenate([t[:, 1:], t[:, :1]], axis=1)
    tr = jnp.concatenate([t[:, -1:], t[:, :-1]], axis=1)
    other = jnp.where(col % 2 == 0, tl, tr)
    o_ref[...] = 1.0 / (1.0 + jnp.exp(other - t))


def _tc_fin(agg3, tro, BD3, b3_t):
    return pl.pallas_call(
        _tc_fin_body,
        grid=(_GRID,),
        in_specs=[_agg_spec(), _rows(16), _full((128, 16)), _full((1, 16))],
        out_specs=_rows(16),
        out_shape=_lin_struct(16),
    )(agg3, tro, BD3, b3_t)


def kernel(z, edge_index, W1_rel, b1, W1_root, W2_rel, b2, W2_root,
           W3_rel, b3, W3_root):
    f32 = jnp.float32
    e_pad = jnp.pad(edge_index, ((0, 0), (0, _E_PAD - _E)),
                    constant_values=_N).reshape(2, _NTILES, _CH, _C)
    z8 = jnp.pad(z.reshape(_N // 8, 512), ((0, _LIN - _N // 8), (0, 0)))

    I8 = jnp.eye(8, dtype=f32)
    kron = jnp.kron
    BD1a = kron(I8, W1_rel[:16].T)
    BD1b = kron(I8, W1_rel[16:].T)
    BDra = kron(I8, W1_root.T[:, :16])
    BDrb = kron(I8, W1_root.T[:, 16:])
    W2T = W2_rel.T
    W2rT = W2_root.T
    BD2a = kron(I8, W2T[:16])
    BD2b = kron(I8, W2T[16:])
    BD2ra = kron(I8, W2rT[:16])
    BD2rb = kron(I8, W2rT[16:])
    BD3 = kron(I8, W3_rel.T)
    BD3r = kron(I8, W3_root.T)
    b1a_t = jnp.tile(b1[:16], 8).reshape(1, 128)
    b1b_t = jnp.tile(b1[16:], 8).reshape(1, 128)
    b2_t = jnp.tile(b2, 8).reshape(1, 128)
    b3_t = jnp.tile(b3, 8).reshape(1, 16)

    sc16 = _make_sc_agg(16)

    m1a, m1b = _tc_pre(z8, BD1a, BD1b)
    agg1a = sc16(m1a.reshape(_N_PAD, 16), e_pad)
    agg1b = sc16(m1b.reshape(_N_PAD, 16), e_pad)
    r1a, r1b = _tc_root(z8, BDra, BDrb)
    m2, r2 = _tc_mid1(agg1a.reshape(2, _LIN, 128), agg1b.reshape(2, _LIN, 128),
                      r1a, r1b, b1a_t, b1b_t, BD2a, BD2b, BD2ra, BD2rb)
    agg2 = sc16(m2.reshape(_N_PAD, 16), e_pad)
    h2 = _tc_mid2(agg2.reshape(2, _LIN, 128), r2, b2_t)
    agg3 = sc16(h2.reshape(_N_PAD, 16), e_pad)
    tro = _tc_finroot(h2, BD3r)
    out = _tc_fin(agg3.reshape(2, _LIN, 128), tro, BD3, b3_t)
    return out[:_N // 8].reshape(_N, 2)


# speedup vs baseline: 1.9609x; 1.0355x over previous
import functools

import jax
import jax.numpy as jnp
from jax import lax
from jax.experimental import pallas as pl
from jax.experimental.pallas import tpu as pltpu
from jax.experimental.pallas import tpu_sc as plsc

_N = 50000
_E = 800000
_N_PAD = 50176
_LIN = _N_PAD * 16 // 128
_BLK = 896
_GRID = _LIN // _BLK
_NTILES = 32
_C = 128
_CH = 196
_K = 14
_E_PAD = _NTILES * _CH * _C
_RPT = _N_PAD // 16
_ZR = 112


@functools.cache
def _make_sc_agg(w):
    mesh = plsc.VectorSubcoreMesh(core_axis_name="c", subcore_axis_name="s")

    @functools.partial(
        pl.kernel,
        out_type=jax.ShapeDtypeStruct((2, _N_PAD, w), jnp.float32),
        mesh=mesh,
        scratch_types=[
            pltpu.VMEM((_CH, _C), jnp.int32),
            pltpu.VMEM((_CH, _C), jnp.int32),
            pltpu.VMEM((_K, _C, w), jnp.float32),
            pltpu.VMEM((_ZR, w), jnp.float32),
            pltpu.VMEM_SHARED((_N_PAD, w), jnp.float32),
            pltpu.SemaphoreType.DMA,
            pltpu.SemaphoreType.DMA,
        ],
        compiler_params=pltpu.CompilerParams(use_tc_tiling_on_sc=False),
    )
    def sc_agg(m_hbm, edges_hbm, out_hbm,
               src_v, dst_v, rows_v, zbuf, acc, gsem, ssem):
        c = lax.axis_index("c")
        s = lax.axis_index("s")
        wid = c * 16 + s
        pltpu.sync_copy(edges_hbm.at[0, wid], src_v)
        pltpu.sync_copy(edges_hbm.at[1, wid], dst_v)
        zv = jnp.zeros((16,), jnp.float32)
        for r in range(_ZR):
            zbuf[r, pl.ds(0, 16)] = zv

        def zero_chunk(j, carry):
            pltpu.sync_copy(zbuf, acc.at[pl.ds(s * _RPT + j * _ZR, _ZR)])
            return carry

        lax.fori_loop(0, _RPT // _ZR, zero_chunk, 0)
        plsc.subcore_barrier()

        def group(g, carry):
            j0 = g * _K
            gathers = []
            for b in range(_K):
                gathers.append(pltpu.async_copy(
                    m_hbm.at[src_v.at[j0 + b]], rows_v.at[b], gsem))
            scatters = []
            for b in range(_K):
                gathers[b].wait()
                scatters.append(pltpu.async_copy(
                    rows_v.at[b], acc.at[dst_v.at[j0 + b]], ssem, add=True))
            for b in range(_K):
                scatters[b].wait()
            return carry

        lax.fori_loop(0, _CH // _K, group, 0)
        plsc.subcore_barrier()
        pltpu.sync_copy(acc.at[pl.ds(s * _RPT, _RPT)],
                        out_hbm.at[c, pl.ds(s * _RPT, _RPT)])

    return sc_agg


def _dot(x, w):
    return lax.dot_general(x, w, (((1,), (0,)), ((), ())),
                           preferred_element_type=jnp.float32)


def _full(shape):
    return pl.BlockSpec(shape, lambda i: (0,) * len(shape))


def _rows(w):
    return pl.BlockSpec((_BLK, w), lambda i: (i, 0))


def _agg_spec():
    return pl.BlockSpec((2, _BLK, 128), lambda i: (0, i, 0))


def _lin_struct(w=128):
    return jax.ShapeDtypeStruct((_LIN, w), jnp.float32)


def _tc_pre_body(z_ref, wa_ref, wb_ref, oa_ref, ob_ref):
    z = z_ref[...]
    oa_ref[...] = _dot(z, wa_ref[...])
    ob_ref[...] = _dot(z, wb_ref[...])


def _tc_pre(z8, BD1a, BD1b):
    return pl.pallas_call(
        _tc_pre_body,
        grid=(_GRID,),
        in_specs=[_rows(512), _full((512, 128)), _full((512, 128))],
        out_specs=[_rows(128), _rows(128)],
        out_shape=[_lin_struct(), _lin_struct()],
    )(z8, BD1a, BD1b)


def _tc_root_body(z_ref, wra_ref, wrb_ref, ra_ref, rb_ref):
    z = z_ref[...]
    ra_ref[...] = _dot(z, wra_ref[...])
    rb_ref[...] = _dot(z, wrb_ref[...])


def _tc_root(z8, BDra, BDrb):
    return pl.pallas_call(
        _tc_root_body,
        grid=(_GRID,),
        in_specs=[_rows(512), _full((512, 128)), _full((512, 128))],
        out_specs=[_rows(128), _rows(128)],
        out_shape=[_lin_struct(), _lin_struct()],
    )(z8, BDra, BDrb)


def _tc_mid1_body(aa_ref, ab_ref, ra_ref, rb_ref, ba_ref, bb_ref,
                  w2a_ref, w2b_ref, w2ra_ref, w2rb_ref, m2_ref, r2_ref):
    ha = jnp.maximum(aa_ref[0] + aa_ref[1] + ba_ref[...] + ra_ref[...], 0.0)
    hb = jnp.maximum(ab_ref[0] + ab_ref[1] + bb_ref[...] + rb_ref[...], 0.0)
    m2_ref[...] = _dot(ha, w2a_ref[...]) + _dot(hb, w2b_ref[...])
    r2_ref[...] = _dot(ha, w2ra_ref[...]) + _dot(hb, w2rb_ref[...])


def _tc_mid1(agg1a, agg1b, r1a, r1b, b1a_t, b1b_t,
             BD2a, BD2b, BD2ra, BD2rb):
    return pl.pallas_call(
        _tc_mid1_body,
        grid=(_GRID,),
        in_specs=[_agg_spec(), _agg_spec(), _rows(128), _rows(128),
                  _full((1, 128)), _full((1, 128)),
                  _full((128, 128)), _full((128, 128)),
                  _full((128, 128)), _full((128, 128))],
        out_specs=[_rows(128), _rows(128)],
        out_shape=[_lin_struct(), _lin_struct()],
    )(agg1a, agg1b, r1a, r1b, b1a_t, b1b_t, BD2a, BD2b, BD2ra, BD2rb)


def _tc_mid2_body(a_ref, r2_ref, b_ref, h2_ref):
    h2_ref[...] = jnp.maximum(a_ref[0] + a_ref[1] + b_ref[...] + r2_ref[...],
                              0.0)


def _tc_mid2(agg2, r2, b2_t):
    return pl.pallas_call(
        _tc_mid2_body,
        grid=(_GRID,),
        in_specs=[_agg_spec(), _rows(128), _full((1, 128))],
        out_specs=_rows(128),
        out_shape=_lin_struct(),
    )(agg2, r2, b2_t)


def _tc_finroot_body(h2_ref, w3r_ref, o_ref):
    o_ref[...] = _dot(h2_ref[...], w3r_ref[...])


def _tc_finroot(h2, BD3r):
    return pl.pallas_call(
        _tc_finroot_body,
        grid=(_GRID,),
        in_specs=[_rows(128), _full((128, 16))],
        out_specs=_rows(16),
        out_shape=_lin_struct(16),
    )(h2, BD3r)


def _tc_fin_body(a_ref, tr_ref, w3_ref, b_ref, o_ref):
    t = _dot(a_ref[0] + a_ref[1], w3_ref[...]) + b_ref[...] + tr_ref[...]
    col = lax.broadcasted_iota(jnp.int32, t.shape, 1)
    tl = jnp.concatenate([t[:, 1:], t[:, :1]], axis=1)
    tr = jnp.concatenate([t[:, -1:], t[:, :-1]], axis=1)
    other = jnp.where(col % 2 == 0, tl, tr)
    o_ref[...] = 1.0 / (1.0 + jnp.exp(other - t))


def _tc_fin(agg3, tro, BD3, b3_t):
    return pl.pallas_call(
        _tc_fin_body,
        grid=(_GRID,),
        in_specs=[_agg_spec(), _rows(16), _full((128, 16)), _full((1, 16))],
        out_specs=_rows(16),
        out_shape=_lin_struct(16),
    )(agg3, tro, BD3, b3_t)


def kernel(z, edge_index, W1_rel, b1, W1_root, W2_rel, b2, W2_root,
           W3_rel, b3, W3_root):
    f32 = jnp.float32
    e_pad = jnp.pad(edge_index, ((0, 0), (0, _E_PAD - _E)),
                    constant_values=_N).reshape(2, _NTILES, _CH, _C)
    z8 = jnp.pad(z.reshape(_N // 8, 512), ((0, _LIN - _N // 8), (0, 0)))

    I8 = jnp.eye(8, dtype=f32)
    kron = jnp.kron
    BD1a = kron(I8, W1_rel[:16].T)
    BD1b = kron(I8, W1_rel[16:].T)
    BDra = kron(I8, W1_root.T[:, :16])
    BDrb = kron(I8, W1_root.T[:, 16:])
    W2T = W2_rel.T
    W2rT = W2_root.T
    BD2a = kron(I8, W2T[:16])
    BD2b = kron(I8, W2T[16:])
    BD2ra = kron(I8, W2rT[:16])
    BD2rb = kron(I8, W2rT[16:])
    BD3 = kron(I8, W3_rel.T)
    BD3r = kron(I8, W3_root.T)
    b1a_t = jnp.tile(b1[:16], 8).reshape(1, 128)
    b1b_t = jnp.tile(b1[16:], 8).reshape(1, 128)
    b2_t = jnp.tile(b2, 8).reshape(1, 128)
    b3_t = jnp.tile(b3, 8).reshape(1, 16)

    sc16 = _make_sc_agg(16)

    m1a, m1b = _tc_pre(z8, BD1a, BD1b)
    agg1a = sc16(m1a.reshape(_N_PAD, 16), e_pad)
    agg1b = sc16(m1b.reshape(_N_PAD, 16), e_pad)
    r1a, r1b = _tc_root(z8, BDra, BDrb)
    m2, r2 = _tc_mid1(agg1a.reshape(2, _LIN, 128), agg1b.reshape(2, _LIN, 128),
                      r1a, r1b, b1a_t, b1b_t, BD2a, BD2b, BD2ra, BD2rb)
    agg2 = sc16(m2.reshape(_N_PAD, 16), e_pad)
    h2 = _tc_mid2(agg2.reshape(2, _LIN, 128), r2, b2_t)
    agg3 = sc16(h2.reshape(_N_PAD, 16), e_pad)
    tro = _tc_finroot(h2, BD3r)
    out = _tc_fin(agg3.reshape(2, _LIN, 128), tro, BD3, b3_t)
    return out[:_N // 8].reshape(_N, 2)
